# baseline (speedup 1.0000x reference)
import jax
import jax.numpy as jnp
from jax import lax
from jax.experimental import pallas as pl

_B, _C, _H, _W = 4, 192, 16, 16
_HW = _H * _W
_P = _B * _HW
_K, _D = 1024, 64


def _vq_body(x_ref, w_ref, b_ref, cb_ref, enc_ref, idx_ref, emb_ref):
    w = w_ref[...]
    cb = cb_ref[...]
    enc = jnp.concatenate(
        [jnp.dot(w, x_ref[b], preferred_element_type=jnp.float32,
                 precision=lax.Precision.DEFAULT) for b in range(_B)],
        axis=1) + b_ref[...]
    cb_h = cb.astype(jnp.bfloat16)
    cb_r = cb - cb_h.astype(jnp.float32)
    cb_m = cb_r.astype(jnp.bfloat16)
    cb_l = (cb_r - cb_m.astype(jnp.float32)).astype(jnp.bfloat16)
    e_h = enc.astype(jnp.bfloat16)
    e_r = enc - e_h.astype(jnp.float32)
    e_m = e_r.astype(jnp.bfloat16)
    e_l = (e_r - e_m.astype(jnp.float32)).astype(jnp.bfloat16)
    cb_x = jnp.concatenate([cb_h, cb_h, cb_h, cb_m, cb_m, cb_l], axis=1)
    e_x = jnp.concatenate([e_h, e_m, e_l, e_h, e_m, e_h], axis=0)
    scores = jnp.dot(cb_x, e_x, preferred_element_type=jnp.float32)
    cnorm2 = 0.5 * jnp.sum(cb * cb, axis=1, keepdims=True)
    negd = scores - cnorm2
    idx = jnp.argmax(negd, axis=0).astype(jnp.int32).reshape(1, _P)
    idx_ref[0] = idx
    kiota = lax.broadcasted_iota(jnp.int32, (_K, _P), 0)
    onehot = (kiota == idx).astype(jnp.bfloat16)
    dn = (((0,), (0,)), ((), ()))
    emb = (lax.dot_general(cb_h, onehot, dn,
                           preferred_element_type=jnp.float32)
           + lax.dot_general(cb_m, onehot, dn,
                             preferred_element_type=jnp.float32))
    for b in range(_B):
        enc_ref[b] = enc[:, b * _HW:(b + 1) * _HW]
        emb_ref[b] = emb[:, b * _HW:(b + 1) * _HW]


def kernel(x, conv_w, conv_b, codebook):
    xr = x.reshape(_B, _C, _HW)
    b2 = conv_b.reshape(_D, 1)
    enc, idx, emb = pl.pallas_call(
        _vq_body,
        in_specs=[
            pl.BlockSpec((_B, _C, _HW), lambda: (0, 0, 0)),
            pl.BlockSpec((_D, _C), lambda: (0, 0)),
            pl.BlockSpec((_D, 1), lambda: (0, 0)),
            pl.BlockSpec((_K, _D), lambda: (0, 0)),
        ],
        out_specs=[
            pl.BlockSpec((_B, _D, _HW), lambda: (0, 0, 0)),
            pl.BlockSpec((1, 1, _P), lambda: (0, 0, 0)),
            pl.BlockSpec((_B, _D, _HW), lambda: (0, 0, 0)),
        ],
        out_shape=[
            jax.ShapeDtypeStruct((_B, _D, _HW), jnp.float32),
            jax.ShapeDtypeStruct((1, 1, _P), jnp.int32),
            jax.ShapeDtypeStruct((_B, _D, _HW), jnp.float32),
        ],
    )(xr, conv_w, b2, codebook)
    emb4 = emb.reshape(_B, _D, _H, _W)
    return (emb4,
            emb4,
            enc.reshape(_B, _D, _H, _W),
            idx.reshape(_B, _H, _W))
